# baseline (speedup 1.0000x reference)
import functools

import jax
import jax.numpy as jnp
import numpy as np
from jax.experimental import pallas as pl
from jax.experimental.pallas import tpu as pltpu

_DIM = 32
_PAIRS = [(a, b) for a in range(_DIM) for b in range(a, _DIM)]
_NTRI = len(_PAIRS)

_RSEL = np.zeros((_DIM, _NTRI), np.float32)
_TSEL = np.zeros((_DIM, _NTRI), np.float32)
_UNPACK = np.zeros((_NTRI, _DIM * _DIM), np.float32)
for _j, (_a, _b) in enumerate(_PAIRS):
    _RSEL[_a, _j] = 1.0
    _TSEL[_b, _j] = 1.0
    _UNPACK[_j, _a * _DIM + _b] = 1.0
    _UNPACK[_j, _b * _DIM + _a] = 1.0


def _kmeans_body(x_ref, c_ref, rsel_ref, tsel_ref, unpack_ref,
                 fill_ref, means_ref, covs_ref, counts_ref, pred_ref,
                 s2_ref, nsteps):
    i = pl.program_id(0)
    x = x_ref[...]
    c = c_ref[...]
    bn = x.shape[0]
    k = c.shape[0]

    xx = jnp.sum(x * x, axis=1, keepdims=True)
    cc = jnp.sum(c * c, axis=1)[None, :]
    xc = jax.lax.dot_general(
        x, c, (((1,), (1,)), ((), ())),
        preferred_element_type=jnp.float32)
    d = jnp.maximum(xx + cc - 2.0 * xc, 0.0)

    iota = jax.lax.broadcasted_iota(jnp.int32, (bn, k), 1)
    rowmin = jnp.min(d, axis=1, keepdims=True)
    cand = jnp.where(d == rowmin, iota, k)
    predc = jnp.min(cand, axis=1, keepdims=True)
    pred_ref[...] = predc.reshape(bn)

    e = jnp.exp(rowmin - d)
    soft = e / jnp.sum(e, axis=1, keepdims=True)
    fill_c = jnp.sum(soft, axis=0)

    p = (iota == predc).astype(jnp.float32)

    counts_c = jnp.sum(p, axis=0)
    sx_c = jax.lax.dot_general(
        p, x, (((0,), (0,)), ((), ())),
        preferred_element_type=jnp.float32)

    xr = jax.lax.dot_general(
        x, rsel_ref[...], (((1,), (0,)), ((), ())),
        preferred_element_type=jnp.float32)
    xt = jax.lax.dot_general(
        x, tsel_ref[...], (((1,), (0,)), ((), ())),
        preferred_element_type=jnp.float32)
    of = xr * xt
    s2_c = jax.lax.dot_general(
        p.astype(jnp.bfloat16), of.astype(jnp.bfloat16),
        (((0,), (0,)), ((), ())),
        preferred_element_type=jnp.float32)

    @pl.when(i == 0)
    def _init():
        fill_ref[...] = fill_c
        counts_ref[...] = counts_c
        means_ref[...] = sx_c
        s2_ref[...] = s2_c

    @pl.when(i > 0)
    def _accum():
        fill_ref[...] += fill_c
        counts_ref[...] += counts_c
        means_ref[...] += sx_c
        s2_ref[...] += s2_c

    @pl.when(i == nsteps - 1)
    def _finalize():
        counts = counts_ref[...]
        safe = jnp.maximum(counts, 1.0)
        inv = 1.0 / safe
        means = means_ref[...] * inv[:, None]
        means_ref[...] = means
        mr = jax.lax.dot_general(
            means, rsel_ref[...], (((1,), (0,)), ((), ())),
            preferred_element_type=jnp.float32)
        mt = jax.lax.dot_general(
            means, tsel_ref[...], (((1,), (0,)), ((), ())),
            preferred_element_type=jnp.float32)
        tri = (s2_ref[...] - counts[:, None] * (mr * mt)) * inv[:, None]
        covs_ref[...] = jax.lax.dot_general(
            tri, unpack_ref[...], (((1,), (0,)), ((), ())),
            preferred_element_type=jnp.float32)
        fill_ref[...] = fill_ref[...] * (1.0 / (nsteps * bn))


@functools.partial(jax.jit, static_argnames=())
def kernel(target, cluster_centers):
    n, dim = target.shape
    k = cluster_centers.shape[0]
    bn = 4096
    nsteps = n // bn

    fill, means, covs_flat, _counts, pred = pl.pallas_call(
        functools.partial(_kmeans_body, nsteps=nsteps),
        grid=(nsteps,),
        in_specs=[
            pl.BlockSpec((bn, dim), lambda i: (i, 0)),
            pl.BlockSpec((k, dim), lambda i: (0, 0)),
            pl.BlockSpec((dim, _NTRI), lambda i: (0, 0)),
            pl.BlockSpec((dim, _NTRI), lambda i: (0, 0)),
            pl.BlockSpec((_NTRI, dim * dim), lambda i: (0, 0)),
        ],
        out_specs=[
            pl.BlockSpec((k,), lambda i: (0,)),
            pl.BlockSpec((k, dim), lambda i: (0, 0)),
            pl.BlockSpec((k, dim * dim), lambda i: (0, 0)),
            pl.BlockSpec((k,), lambda i: (0,)),
            pl.BlockSpec((bn,), lambda i: (i,)),
        ],
        out_shape=[
            jax.ShapeDtypeStruct((k,), jnp.float32),
            jax.ShapeDtypeStruct((k, dim), jnp.float32),
            jax.ShapeDtypeStruct((k, dim * dim), jnp.float32),
            jax.ShapeDtypeStruct((k,), jnp.float32),
            jax.ShapeDtypeStruct((n,), jnp.int32),
        ],
        scratch_shapes=[pltpu.VMEM((k, _NTRI), jnp.float32)],
    )(target, cluster_centers,
      jnp.asarray(_RSEL), jnp.asarray(_TSEL), jnp.asarray(_UNPACK))

    return fill, means, covs_flat.reshape(k, dim, dim), pred


# speedup vs baseline: 1.0863x; 1.0863x over previous
import functools

import jax
import jax.numpy as jnp
import numpy as np
from jax.experimental import pallas as pl
from jax.experimental.pallas import tpu as pltpu

_DIM = 32
_PAIRS = [(a, b) for a in range(_DIM) for b in range(a, _DIM)]
_NTRI = len(_PAIRS)

_RSEL = np.zeros((_DIM, _NTRI), np.float32)
_TSEL = np.zeros((_DIM, _NTRI), np.float32)
_UNPACK = np.zeros((_NTRI, _DIM * _DIM), np.float32)
for _j, (_a, _b) in enumerate(_PAIRS):
    _RSEL[_a, _j] = 1.0
    _TSEL[_b, _j] = 1.0
    _UNPACK[_j, _a * _DIM + _b] = 1.0
    _UNPACK[_j, _b * _DIM + _a] = 1.0


def _kmeans_body(x_ref, c_ref, rsel_ref, tsel_ref, unpack_ref,
                 fill_ref, means_ref, covs_ref, counts_ref, pred_ref,
                 s2_ref, nsteps):
    i = pl.program_id(0)
    x = x_ref[...]
    c = c_ref[...]
    bn = x.shape[0]
    k = c.shape[0]

    xx = jnp.sum(x * x, axis=1, keepdims=True)
    cc = jnp.sum(c * c, axis=1)[None, :]
    xc = jax.lax.dot_general(
        x, c, (((1,), (1,)), ((), ())),
        preferred_element_type=jnp.float32)
    d = xx + cc - 2.0 * xc

    pred = jnp.argmin(d, axis=1).astype(jnp.int32)
    pred_ref[...] = pred

    rowmin = jnp.min(d, axis=1, keepdims=True)
    e = jnp.exp(rowmin - d)
    soft = e / jnp.sum(e, axis=1, keepdims=True)
    fill_c = jnp.sum(soft, axis=0)

    iota = jax.lax.broadcasted_iota(jnp.int32, (bn, k), 1)
    p = (iota == pred[:, None]).astype(jnp.bfloat16)

    dimn = x.shape[1]
    xa = jnp.concatenate(
        [x, jnp.ones((bn, 1), jnp.float32)], axis=1).astype(jnp.bfloat16)
    sxa = jax.lax.dot_general(
        p, xa, (((0,), (0,)), ((), ())),
        preferred_element_type=jnp.float32)
    sx_c = sxa[:, :dimn]
    counts_c = sxa[:, dimn]

    xr = jax.lax.dot_general(
        x, rsel_ref[...], (((1,), (0,)), ((), ())),
        preferred_element_type=jnp.float32)
    xt = jax.lax.dot_general(
        x, tsel_ref[...], (((1,), (0,)), ((), ())),
        preferred_element_type=jnp.float32)
    of = xr * xt
    s2_c = jax.lax.dot_general(
        p, of.astype(jnp.bfloat16),
        (((0,), (0,)), ((), ())),
        preferred_element_type=jnp.float32)

    @pl.when(i == 0)
    def _init():
        fill_ref[...] = fill_c
        counts_ref[...] = counts_c
        means_ref[...] = sx_c
        s2_ref[...] = s2_c

    @pl.when(i > 0)
    def _accum():
        fill_ref[...] += fill_c
        counts_ref[...] += counts_c
        means_ref[...] += sx_c
        s2_ref[...] += s2_c

    @pl.when(i == nsteps - 1)
    def _finalize():
        counts = counts_ref[...]
        safe = jnp.maximum(counts, 1.0)
        inv = 1.0 / safe
        means = means_ref[...] * inv[:, None]
        means_ref[...] = means
        mr = jax.lax.dot_general(
            means, rsel_ref[...], (((1,), (0,)), ((), ())),
            preferred_element_type=jnp.float32)
        mt = jax.lax.dot_general(
            means, tsel_ref[...], (((1,), (0,)), ((), ())),
            preferred_element_type=jnp.float32)
        tri = (s2_ref[...] - counts[:, None] * (mr * mt)) * inv[:, None]
        covs_ref[...] = jax.lax.dot_general(
            tri, unpack_ref[...], (((1,), (0,)), ((), ())),
            preferred_element_type=jnp.float32)
        fill_ref[...] = fill_ref[...] * (1.0 / (nsteps * bn))


@functools.partial(jax.jit, static_argnames=())
def kernel(target, cluster_centers):
    n, dim = target.shape
    k = cluster_centers.shape[0]
    bn = 4096
    nsteps = n // bn

    fill, means, covs_flat, _counts, pred = pl.pallas_call(
        functools.partial(_kmeans_body, nsteps=nsteps),
        grid=(nsteps,),
        in_specs=[
            pl.BlockSpec((bn, dim), lambda i: (i, 0)),
            pl.BlockSpec((k, dim), lambda i: (0, 0)),
            pl.BlockSpec((dim, _NTRI), lambda i: (0, 0)),
            pl.BlockSpec((dim, _NTRI), lambda i: (0, 0)),
            pl.BlockSpec((_NTRI, dim * dim), lambda i: (0, 0)),
        ],
        out_specs=[
            pl.BlockSpec((k,), lambda i: (0,)),
            pl.BlockSpec((k, dim), lambda i: (0, 0)),
            pl.BlockSpec((k, dim * dim), lambda i: (0, 0)),
            pl.BlockSpec((k,), lambda i: (0,)),
            pl.BlockSpec((bn,), lambda i: (i,)),
        ],
        out_shape=[
            jax.ShapeDtypeStruct((k,), jnp.float32),
            jax.ShapeDtypeStruct((k, dim), jnp.float32),
            jax.ShapeDtypeStruct((k, dim * dim), jnp.float32),
            jax.ShapeDtypeStruct((k,), jnp.float32),
            jax.ShapeDtypeStruct((n,), jnp.int32),
        ],
        scratch_shapes=[pltpu.VMEM((k, _NTRI), jnp.float32)],
    )(target, cluster_centers,
      jnp.asarray(_RSEL), jnp.asarray(_TSEL), jnp.asarray(_UNPACK))

    return fill, means, covs_flat.reshape(k, dim, dim), pred


# speedup vs baseline: 1.1232x; 1.0339x over previous
import functools

import jax
import jax.numpy as jnp
import numpy as np
from jax.experimental import pallas as pl
from jax.experimental.pallas import tpu as pltpu

_DIM = 32
_PAIRS = [(a, b) for a in range(_DIM) for b in range(a, _DIM)]
_NTRI = len(_PAIRS)

_RSEL = np.zeros((_DIM, _NTRI), np.float32)
_TSEL = np.zeros((_DIM, _NTRI), np.float32)
_UNPACK = np.zeros((_NTRI, _DIM * _DIM), np.float32)
for _j, (_a, _b) in enumerate(_PAIRS):
    _RSEL[_a, _j] = 1.0
    _TSEL[_b, _j] = 1.0
    _UNPACK[_j, _a * _DIM + _b] = 1.0
    _UNPACK[_j, _b * _DIM + _a] = 1.0


def _kmeans_body(x_ref, c_ref, rsel_ref, tsel_ref, unpack_ref,
                 fill_ref, means_ref, covs_ref, counts_ref, pred_ref,
                 s2_ref, nsteps):
    i = pl.program_id(0)
    x = x_ref[...]
    c = c_ref[...]
    bn = x.shape[0]
    k = c.shape[0]

    xx = jnp.sum(x * x, axis=1, keepdims=True)
    cc = jnp.sum(c * c, axis=1, keepdims=True)
    onen = jnp.ones((bn, 1), jnp.float32)
    onek = jnp.ones((k, 1), jnp.float32)
    xa = jnp.concatenate([x, onen, xx], axis=1)
    ca = jnp.concatenate([c * (-2.0), cc, onek], axis=1)
    d = jax.lax.dot_general(
        xa, ca, (((1,), (1,)), ((), ())),
        preferred_element_type=jnp.float32)

    pred = jnp.argmin(d, axis=1).astype(jnp.int32)
    pred_ref[...] = pred

    rowmin = jnp.min(d, axis=1, keepdims=True)
    e = jnp.exp(rowmin - d)
    soft = e / jnp.sum(e, axis=1, keepdims=True)
    fill_c = jnp.sum(soft, axis=0)

    iota = jax.lax.broadcasted_iota(jnp.int32, (bn, k), 1)
    p = (iota == pred[:, None]).astype(jnp.float32)

    dimn = x.shape[1]
    sxa = jax.lax.dot_general(
        p, xa, (((0,), (0,)), ((), ())),
        preferred_element_type=jnp.float32)
    sx_c = sxa[:, :dimn]
    counts_c = sxa[:, dimn]

    xr = jax.lax.dot_general(
        x, rsel_ref[...], (((1,), (0,)), ((), ())),
        preferred_element_type=jnp.float32)
    xt = jax.lax.dot_general(
        x, tsel_ref[...], (((1,), (0,)), ((), ())),
        preferred_element_type=jnp.float32)
    of = xr * xt
    s2_c = jax.lax.dot_general(
        p.astype(jnp.bfloat16), of.astype(jnp.bfloat16),
        (((0,), (0,)), ((), ())),
        preferred_element_type=jnp.float32)

    @pl.when(i == 0)
    def _init():
        fill_ref[...] = fill_c
        counts_ref[...] = counts_c
        means_ref[...] = sx_c
        s2_ref[...] = s2_c

    @pl.when(i > 0)
    def _accum():
        fill_ref[...] += fill_c
        counts_ref[...] += counts_c
        means_ref[...] += sx_c
        s2_ref[...] += s2_c

    @pl.when(i == nsteps - 1)
    def _finalize():
        counts = counts_ref[...]
        safe = jnp.maximum(counts, 1.0)
        inv = 1.0 / safe
        means = means_ref[...] * inv[:, None]
        means_ref[...] = means
        mr = jax.lax.dot_general(
            means, rsel_ref[...], (((1,), (0,)), ((), ())),
            preferred_element_type=jnp.float32)
        mt = jax.lax.dot_general(
            means, tsel_ref[...], (((1,), (0,)), ((), ())),
            preferred_element_type=jnp.float32)
        tri = (s2_ref[...] - counts[:, None] * (mr * mt)) * inv[:, None]
        covs_ref[...] = jax.lax.dot_general(
            tri, unpack_ref[...], (((1,), (0,)), ((), ())),
            preferred_element_type=jnp.float32)
        fill_ref[...] = fill_ref[...] * (1.0 / (nsteps * bn))


@functools.partial(jax.jit, static_argnames=())
def kernel(target, cluster_centers):
    n, dim = target.shape
    k = cluster_centers.shape[0]
    bn = 4096
    nsteps = n // bn

    fill, means, covs_flat, _counts, pred = pl.pallas_call(
        functools.partial(_kmeans_body, nsteps=nsteps),
        grid=(nsteps,),
        in_specs=[
            pl.BlockSpec((bn, dim), lambda i: (i, 0)),
            pl.BlockSpec((k, dim), lambda i: (0, 0)),
            pl.BlockSpec((dim, _NTRI), lambda i: (0, 0)),
            pl.BlockSpec((dim, _NTRI), lambda i: (0, 0)),
            pl.BlockSpec((_NTRI, dim * dim), lambda i: (0, 0)),
        ],
        out_specs=[
            pl.BlockSpec((k,), lambda i: (0,)),
            pl.BlockSpec((k, dim), lambda i: (0, 0)),
            pl.BlockSpec((k, dim * dim), lambda i: (0, 0)),
            pl.BlockSpec((k,), lambda i: (0,)),
            pl.BlockSpec((bn,), lambda i: (i,)),
        ],
        out_shape=[
            jax.ShapeDtypeStruct((k,), jnp.float32),
            jax.ShapeDtypeStruct((k, dim), jnp.float32),
            jax.ShapeDtypeStruct((k, dim * dim), jnp.float32),
            jax.ShapeDtypeStruct((k,), jnp.float32),
            jax.ShapeDtypeStruct((n,), jnp.int32),
        ],
        scratch_shapes=[pltpu.VMEM((k, _NTRI), jnp.float32)],
    )(target, cluster_centers,
      jnp.asarray(_RSEL), jnp.asarray(_TSEL), jnp.asarray(_UNPACK))

    return fill, means, covs_flat.reshape(k, dim, dim), pred
